# Initial kernel scaffold; baseline (speedup 1.0000x reference)
import jax
import jax.numpy as jnp
from jax.experimental import pallas as pl


def kernel(x, dia_len, qmask):
    raise NotImplementedError("write your pallas kernel here")


# speedup vs baseline: 2.6506x; 2.6506x over previous
import math

import jax
import jax.numpy as jnp
from jax.experimental import pallas as pl
from jax.experimental.pallas import tpu as pltpu

WINDOW = 15
S = 512
D = 256
NSPK = 9

_ACOS_C = (
    1.5707963050,
    -0.2145988016,
    0.0889789874,
    -0.0501743046,
    0.0308918810,
    -0.0170881256,
    0.0066700901,
    -0.0012624911,
)


def _acos(x):
    ax = jnp.abs(x)
    p = jnp.float32(_ACOS_C[7])
    for c in _ACOS_C[6::-1]:
        p = p * ax + jnp.float32(c)
    r = jnp.sqrt(jnp.maximum(1.0 - ax, 0.0)) * p
    return jnp.where(x >= 0.0, r, jnp.float32(math.pi) - r)


def _adj_kernel(dia_ref, x_ref, q_ref, out_ref):
    b = pl.program_id(0)
    xb = x_ref[0]
    nrm = jnp.sqrt(jnp.sum(xb * xb, axis=1, keepdims=True))
    xn = xb / jnp.maximum(nrm, 1e-8)

    cos = jax.lax.dot_general(
        xn, xn, (((1,), (1,)), ((), ())), preferred_element_type=jnp.float32
    )
    cos = jnp.clip(cos, -1.0, 1.0)
    w = 1.0 - _acos(cos) * (1.0 / math.pi)

    ii = jax.lax.broadcasted_iota(jnp.int32, (S, S), 0)
    jj = jax.lax.broadcasted_iota(jnp.int32, (S, S), 1)
    dl = dia_ref[b]
    winm = (jnp.abs(ii - jj) <= WINDOW) & (ii < dl) & (jj < dl)

    q = q_ref[0]
    qmax = jnp.max(q, axis=1, keepdims=True)
    io = jax.lax.broadcasted_iota(jnp.int32, (S, NSPK), 1)
    spk = jnp.min(jnp.where(q >= qmax, io, NSPK), axis=1)
    same = spk[:, None] == spk[None, :]
    spkm = winm & same

    cnt = jnp.sum(spkm.astype(jnp.float32), axis=1)
    adj = w * winm.astype(jnp.float32) + w * spkm.astype(jnp.float32) * (
        cnt > 1.0
    ).astype(jnp.float32)[:, None]

    deg = jnp.sum(adj, axis=1)
    deg = jnp.where(deg == 0.0, 1.0, deg)
    dinv = jax.lax.rsqrt(deg)
    out_ref[0] = adj * dinv[:, None] * dinv[None, :]


def kernel(x, dia_len, qmask):
    B = x.shape[0]
    qt = jnp.transpose(qmask, (1, 0, 2))
    dl = dia_len.astype(jnp.int32)
    grid_spec = pltpu.PrefetchScalarGridSpec(
        num_scalar_prefetch=1,
        grid=(B,),
        in_specs=[
            pl.BlockSpec((1, S, D), lambda b, d: (b, 0, 0)),
            pl.BlockSpec((1, S, NSPK), lambda b, d: (b, 0, 0)),
        ],
        out_specs=pl.BlockSpec((1, S, S), lambda b, d: (b, 0, 0)),
    )
    return pl.pallas_call(
        _adj_kernel,
        grid_spec=grid_spec,
        out_shape=jax.ShapeDtypeStruct((B, S, S), jnp.float32),
    )(dl, x, qt)


# speedup vs baseline: 3.2921x; 1.2420x over previous
import math

import jax
import jax.numpy as jnp
from jax.experimental import pallas as pl
from jax.experimental.pallas import tpu as pltpu

WINDOW = 15
S = 512
D = 256
NSPK = 9
T = 128
NT = S // T

_ACOS_C = (1.5707288, -0.2121144, 0.0742610, -0.0187293)


def _wfun(cos):
    ax = jnp.abs(cos)
    p = jnp.float32(_ACOS_C[3])
    for c in _ACOS_C[2::-1]:
        p = p * ax + jnp.float32(c)
    r = jnp.sqrt(jnp.maximum(1.0 - ax, 0.0)) * p * jnp.float32(1.0 / math.pi)
    return jnp.where(cos >= 0.0, 1.0 - r, r)


def _adj_kernel(dia_ref, x_ref, q_ref, out_ref):
    b = pl.program_id(0)
    dl = dia_ref[b]
    xb = x_ref[0]
    xn = xb * jax.lax.rsqrt(
        jnp.maximum(jnp.sum(xb * xb, axis=1, keepdims=True), 1e-16)
    )

    q = q_ref[0]
    qmax = jnp.max(q, axis=1, keepdims=True)
    io = jax.lax.broadcasted_iota(jnp.int32, (S, NSPK), 1)
    spk = jnp.min(jnp.where(q >= qmax, io, NSPK), axis=1)

    dinv_parts = []
    for ti in range(NT):
        r0 = ti * T
        xr = xn[r0 : r0 + T]
        spk_r = spk[r0 : r0 + T]
        row_ii = jax.lax.broadcasted_iota(jnp.int32, (T, T), 0) + r0
        col_jj = jax.lax.broadcasted_iota(jnp.int32, (T, T), 1)
        tjs = [tj for tj in (ti - 1, ti, ti + 1) if 0 <= tj < NT]
        tiles = []
        cnt = jnp.zeros((T,), jnp.float32)
        for tj in tjs:
            c0 = tj * T
            cos = jax.lax.dot_general(
                xr,
                xn[c0 : c0 + T],
                (((1,), (1,)), ((), ())),
                preferred_element_type=jnp.float32,
            )
            w = _wfun(cos)
            jj = col_jj + c0
            winm = (
                (jnp.abs(row_ii - jj) <= WINDOW) & (row_ii < dl) & (jj < dl)
            )
            samet = spk_r[:, None] == spk[c0 : c0 + T][None, :]
            spkf = (winm & samet).astype(jnp.float32)
            winf = winm.astype(jnp.float32)
            cnt = cnt + jnp.sum(spkf, axis=1)
            tiles.append((c0, w, winf, spkf))
        gate = (cnt > 1.0).astype(jnp.float32)[:, None]
        deg = jnp.zeros((T,), jnp.float32)
        for c0, w, winf, spkf in tiles:
            pre = w * (winf + spkf * gate)
            deg = deg + jnp.sum(pre, axis=1)
            out_ref[0, r0 : r0 + T, c0 : c0 + T] = pre
        lo = tjs[0] * T
        hi = (tjs[-1] + 1) * T
        if lo > 0:
            out_ref[0, r0 : r0 + T, 0:lo] = jnp.zeros((T, lo), jnp.float32)
        if hi < S:
            out_ref[0, r0 : r0 + T, hi:S] = jnp.zeros((T, S - hi), jnp.float32)
        dinv_parts.append(jax.lax.rsqrt(jnp.where(deg == 0.0, 1.0, deg)))

    for ti in range(NT):
        r0 = ti * T
        dr = dinv_parts[ti][:, None]
        for tj in (ti - 1, ti, ti + 1):
            if not (0 <= tj < NT):
                continue
            c0 = tj * T
            dc = dinv_parts[tj][None, :]
            out_ref[0, r0 : r0 + T, c0 : c0 + T] = (
                out_ref[0, r0 : r0 + T, c0 : c0 + T] * dr * dc
            )


def kernel(x, dia_len, qmask):
    B = x.shape[0]
    qt = jnp.transpose(qmask, (1, 0, 2))
    dl = dia_len.astype(jnp.int32)
    grid_spec = pltpu.PrefetchScalarGridSpec(
        num_scalar_prefetch=1,
        grid=(B,),
        in_specs=[
            pl.BlockSpec((1, S, D), lambda b, d: (b, 0, 0)),
            pl.BlockSpec((1, S, NSPK), lambda b, d: (b, 0, 0)),
        ],
        out_specs=pl.BlockSpec((1, S, S), lambda b, d: (b, 0, 0)),
    )
    return pl.pallas_call(
        _adj_kernel,
        grid_spec=grid_spec,
        out_shape=jax.ShapeDtypeStruct((B, S, S), jnp.float32),
    )(dl, x, qt)
